# baseline (device time: 51227 ns/iter reference)
import jax
import jax.numpy as jnp
from jax import lax
from jax.experimental import pallas as pl
from jax.experimental.pallas import tpu as pltpu

N_DEV = 4
NP = 2


def kernel(A, B, stage="full"):
    m, _ = A.shape
    _, n = B.shape
    chunk = m // N_DEV
    half = n // 2
    piece = chunk // NP

    def body(
        a_ref,
        b_ref,
        out_ref,
        b_bf,
        pcL, pcR,
        chainL, chainR, dirL, dirR, sumL, sumR, msendL, msendR,
        e1ss, e1rs, e2ss, e2rs, e3ss, e3rs, e4ss, e4rs,
        m1ss, m1rs, m2ss, m2rs,
        ag_ssem_r, ag_rsem_r, ag_ssem_l, ag_rsem_l,
    ):
        my = lax.axis_index("i")
        left = (my - 1) % N_DEV
        right = (my + 1) % N_DEV

        barrier_sem = pltpu.get_barrier_semaphore()
        for nbr in (left, right):
            pl.semaphore_signal(
                barrier_sem,
                inc=1,
                device_id=(nbr,),
                device_id_type=pl.DeviceIdType.MESH,
            )
        pl.semaphore_wait(barrier_sem, 2)

        def rdma(src, dst, ssem, rsem, target):
            return pltpu.make_async_remote_copy(
                src_ref=src, dst_ref=dst, send_sem=ssem, recv_sem=rsem,
                device_id=(target,), device_id_type=pl.DeviceIdType.MESH,
            )

        def rowsl(ref, c, p):
            return ref.at[pl.ds(c * chunk + p * piece, piece), :]

        E1 = [rdma(rowsl(pcL, (my + 2) % N_DEV, p), chainL.at[p],
                   e1ss.at[p], e1rs.at[p], right) for p in range(NP)]
        E2 = [rdma(rowsl(pcR, (my + 2) % N_DEV, p), chainR.at[p],
                   e2ss.at[p], e2rs.at[p], left) for p in range(NP)]
        E3 = [rdma(rowsl(pcL, (my - 1) % N_DEV, p), dirL.at[p],
                   e3ss.at[p], e3rs.at[p], left) for p in range(NP)]
        E4 = [rdma(rowsl(pcR, (my + 1) % N_DEV, p), dirR.at[p],
                   e4ss.at[p], e4rs.at[p], right) for p in range(NP)]
        M1 = [rdma(msendL.at[p], sumL.at[p],
                   m1ss.at[p], m1rs.at[p], right) for p in range(NP)]
        M2 = [rdma(msendR.at[p], sumR.at[p],
                   m2ss.at[p], m2rs.at[p], left) for p in range(NP)]

        def out_sl(c, p, col0):
            return out_ref.at[
                pl.ds(c * chunk + p * piece, piece), pl.ds(col0, half)
            ]

        ag_send_r = [[rdma(out_sl((my - s) % N_DEV, p, 0),
                           out_sl((my - s) % N_DEV, p, 0),
                           ag_ssem_r.at[s, p], ag_rsem_r.at[s, p], right)
                      for p in range(NP)] for s in range(3)]
        ag_recv_r = [[rdma(out_sl((my - 1 - s) % N_DEV, p, 0),
                           out_sl((my - 1 - s) % N_DEV, p, 0),
                           ag_ssem_r.at[s, p], ag_rsem_r.at[s, p], right)
                      for p in range(NP)] for s in range(3)]
        ag_send_l = [[rdma(out_sl((my + s) % N_DEV, p, half),
                           out_sl((my + s) % N_DEV, p, half),
                           ag_ssem_l.at[s, p], ag_rsem_l.at[s, p], left)
                      for p in range(NP)] for s in range(3)]
        ag_recv_l = [[rdma(out_sl((my + 1 + s) % N_DEV, p, half),
                           out_sl((my + 1 + s) % N_DEV, p, half),
                           ag_ssem_l.at[s, p], ag_rsem_l.at[s, p], left)
                      for p in range(NP)] for s in range(3)]

        def dotL(c, p):
            ap = a_ref[pl.ds(c * chunk + p * piece, piece), :].astype(
                jnp.bfloat16
            )
            pcL[pl.ds(c * chunk + p * piece, piece), :] = jnp.dot(
                ap, b_bf[:, :half], preferred_element_type=jnp.float32
            ).astype(jnp.bfloat16)

        def dotR(c, p):
            ap = a_ref[pl.ds(c * chunk + p * piece, piece), :].astype(
                jnp.bfloat16
            )
            pcR[pl.ds(c * chunk + p * piece, piece), :] = jnp.dot(
                ap, b_bf[:, half:], preferred_element_type=jnp.float32
            ).astype(jnp.bfloat16)

        comm = stage != "mm"

        b_bf[:, :half] = b_ref[:, :half].astype(jnp.bfloat16)
        for p in range(NP):
            dotL((my + 2) % N_DEV, p)
            if comm:
                E1[p].start()
        b_bf[:, half:] = b_ref[:, half:].astype(jnp.bfloat16)
        for p in range(NP):
            dotR((my + 2) % N_DEV, p)
            if comm:
                E2[p].start()
        for p in range(NP):
            dotR((my + 1) % N_DEV, p)
            if comm:
                E4[p].start()
        for p in range(NP):
            dotL((my - 1) % N_DEV, p)
            if comm:
                E3[p].start()

        for p in range(NP):
            dotL((my + 1) % N_DEV, p)
            if comm:
                E1[p].wait_recv()
                msendL[p, :, :] = (
                    pcL[pl.ds(((my + 1) % N_DEV) * chunk + p * piece, piece), :]
                    + chainL[p, :, :]
                )
                M1[p].start()
        for p in range(NP):
            dotR((my - 1) % N_DEV, p)
            if comm:
                E2[p].wait_recv()
                msendR[p, :, :] = (
                    pcR[pl.ds(((my - 1) % N_DEV) * chunk + p * piece, piece), :]
                    + chainR[p, :, :]
                )
                M2[p].start()

        for p in range(NP):
            dotL(my, p)
            if comm:
                M1[p].wait_recv()
                E3[p].wait_recv()
                zL = (
                    pcL[pl.ds(my * chunk + p * piece, piece), :].astype(
                        jnp.float32
                    )
                    + sumL[p, :, :].astype(jnp.float32)
                    + dirL[p, :, :].astype(jnp.float32)
                )
                out_ref[pl.ds(my * chunk + p * piece, piece), pl.ds(0, half)] = (
                    zL / (1.0 + jnp.exp(-zL))
                ).astype(jnp.bfloat16)
                if stage == "full":
                    ag_send_r[0][p].start()
        for p in range(NP):
            dotR(my, p)
            if comm:
                M2[p].wait_recv()
                E4[p].wait_recv()
                zR = (
                    pcR[pl.ds(my * chunk + p * piece, piece), :].astype(
                        jnp.float32
                    )
                    + sumR[p, :, :].astype(jnp.float32)
                    + dirR[p, :, :].astype(jnp.float32)
                )
                out_ref[
                    pl.ds(my * chunk + p * piece, piece), pl.ds(half, half)
                ] = (zR / (1.0 + jnp.exp(-zR))).astype(jnp.bfloat16)
                if stage == "full":
                    ag_send_l[0][p].start()

        if stage == "mm":
            return

        if stage == "full":
            for s in range(3):
                for p in range(NP):
                    ag_recv_r[s][p].wait_recv()
                    if s < 2:
                        ag_send_r[s + 1][p].start()
                    ag_recv_l[s][p].wait_recv()
                    if s < 2:
                        ag_send_l[s + 1][p].start()

        for flow in (E1, E2, E3, E4, M1, M2):
            for op in flow:
                op.wait_send()
        if stage == "full":
            for grid in (ag_send_r, ag_send_l):
                for ops in grid:
                    for op in ops:
                        op.wait_send()

    return pl.pallas_call(
        body,
        out_shape=jax.ShapeDtypeStruct((m, n), jnp.bfloat16),
        in_specs=[
            pl.BlockSpec(memory_space=pltpu.VMEM),
            pl.BlockSpec(memory_space=pltpu.VMEM),
        ],
        out_specs=pl.BlockSpec(memory_space=pltpu.VMEM),
        scratch_shapes=[
            pltpu.VMEM(B.shape, jnp.bfloat16),
            pltpu.VMEM((m, half), jnp.bfloat16),
            pltpu.VMEM((m, half), jnp.bfloat16),
            pltpu.VMEM((NP, piece, half), jnp.bfloat16),
            pltpu.VMEM((NP, piece, half), jnp.bfloat16),
            pltpu.VMEM((NP, piece, half), jnp.bfloat16),
            pltpu.VMEM((NP, piece, half), jnp.bfloat16),
            pltpu.VMEM((NP, piece, half), jnp.bfloat16),
            pltpu.VMEM((NP, piece, half), jnp.bfloat16),
            pltpu.VMEM((NP, piece, half), jnp.bfloat16),
            pltpu.VMEM((NP, piece, half), jnp.bfloat16),
            pltpu.SemaphoreType.DMA((NP,)),
            pltpu.SemaphoreType.DMA((NP,)),
            pltpu.SemaphoreType.DMA((NP,)),
            pltpu.SemaphoreType.DMA((NP,)),
            pltpu.SemaphoreType.DMA((NP,)),
            pltpu.SemaphoreType.DMA((NP,)),
            pltpu.SemaphoreType.DMA((NP,)),
            pltpu.SemaphoreType.DMA((NP,)),
            pltpu.SemaphoreType.DMA((NP,)),
            pltpu.SemaphoreType.DMA((NP,)),
            pltpu.SemaphoreType.DMA((NP,)),
            pltpu.SemaphoreType.DMA((NP,)),
            pltpu.SemaphoreType.DMA((3, NP)),
            pltpu.SemaphoreType.DMA((3, NP)),
            pltpu.SemaphoreType.DMA((3, NP)),
            pltpu.SemaphoreType.DMA((3, NP)),
        ],
        compiler_params=pltpu.CompilerParams(collective_id=0),
    )(A, B)
